# baseline (speedup 1.0000x reference)
import functools
import math

import jax
import jax.numpy as jnp
from jax.experimental import pallas as pl
from jax.experimental.pallas import tpu as pltpu


def _attn_kernel(q_ref, k_ref, v_ref, o_ref, *, nsub):
    bh = q_ref.shape[0]
    tq = q_ref.shape[1]
    sub = tq // nsub
    d = q_ref.shape[-1]
    scale = jnp.float32((1.0 / math.sqrt(d)) * math.log2(math.e))

    for h in range(bh):
        k = k_ref[h]
        v = v_ref[h]

        for r in range(nsub):
            qr = q_ref[h, pl.ds(r * sub, sub), :]
            s = jax.lax.dot_general(qr, k, (((1,), (1,)), ((), ())),
                                    preferred_element_type=jnp.float32)
            m = s.max(axis=-1, keepdims=True)
            p = jnp.exp2((s - m) * scale)
            l = p.sum(axis=-1, keepdims=True)
            ot = jax.lax.dot_general(v, p.astype(jnp.bfloat16),
                                     (((0,), (1,)), ((), ())),
                                     preferred_element_type=jnp.float32)
            ot = ot * (1.0 / l).reshape(1, sub)
            o_ref[h, pl.ds(r * sub, sub), :] = ot.T.astype(o_ref.dtype)


def _attention(q, k, v, *, block_h=4, nsub=8):
    B, H, S, D = q.shape
    BH = B * H

    qr = q.reshape(BH, S, D).astype(jnp.bfloat16)
    kr = k.reshape(BH, S, D).astype(jnp.bfloat16)
    vr = v.reshape(BH, S, D).astype(jnp.bfloat16)

    bh = block_h
    while BH % bh:
        bh -= 1

    spec = pl.BlockSpec((bh, S, D), lambda g: (g, 0, 0))

    out = pl.pallas_call(
        functools.partial(_attn_kernel, nsub=nsub),
        out_shape=jax.ShapeDtypeStruct((BH, S, D), q.dtype),
        grid=(BH // bh,),
        in_specs=[spec, spec, spec],
        out_specs=spec,
        compiler_params=pltpu.CompilerParams(
            dimension_semantics=("parallel",),
            vmem_limit_bytes=100 * 1024 * 1024,
        ),
    )(qr, kr, vr)
    return out.reshape(B, H, S, D)


def kernel(q, k, v, mask):
    del mask
    return _attention(q, k, v)


# speedup vs baseline: 1.2327x; 1.2327x over previous
import functools
import math

import jax
import jax.numpy as jnp
from jax.experimental import pallas as pl
from jax.experimental.pallas import tpu as pltpu


def _attn_kernel(q_ref, k_ref, v_ref, o_ref, *, nsub):
    bh = q_ref.shape[0]
    tq = q_ref.shape[1]
    sub = tq // nsub
    d = q_ref.shape[-1]
    scale = jnp.float32((1.0 / math.sqrt(d)) * math.log2(math.e))

    for h in range(bh):
        k = k_ref[h].astype(jnp.bfloat16)
        v = v_ref[h].astype(jnp.bfloat16)

        for r in range(nsub):
            qr = q_ref[h, pl.ds(r * sub, sub), :].astype(jnp.bfloat16)
            s = jax.lax.dot_general(qr, k, (((1,), (1,)), ((), ())),
                                    preferred_element_type=jnp.float32)
            m = s.max(axis=-1, keepdims=True)
            p = jnp.exp2((s - m) * scale)
            l = p.sum(axis=-1, keepdims=True)
            ot = jax.lax.dot_general(v, p.astype(jnp.bfloat16),
                                     (((0,), (1,)), ((), ())),
                                     preferred_element_type=jnp.float32)
            o_ref[h, pl.ds(r * sub, sub), :] = (
                ot.T * (1.0 / l)).astype(o_ref.dtype)


def _attention(q, k, v, *, block_h=2, nsub=8):
    B, H, S, D = q.shape
    BH = B * H

    qr = q.reshape(BH, S, D)
    kr = k.reshape(BH, S, D)
    vr = v.reshape(BH, S, D)

    bh = block_h
    while BH % bh:
        bh -= 1

    spec = pl.BlockSpec((bh, S, D), lambda g: (g, 0, 0))

    out = pl.pallas_call(
        functools.partial(_attn_kernel, nsub=nsub),
        out_shape=jax.ShapeDtypeStruct((BH, S, D), q.dtype),
        grid=(BH // bh,),
        in_specs=[spec, spec, spec],
        out_specs=spec,
        compiler_params=pltpu.CompilerParams(
            dimension_semantics=("parallel",),
            vmem_limit_bytes=100 * 1024 * 1024,
        ),
    )(qr, kr, vr)
    return out.reshape(B, H, S, D)


def kernel(q, k, v, mask):
    del mask
    return _attention(q, k, v)
